# baseline (speedup 1.0000x reference)
import functools

import jax
import jax.numpy as jnp
from jax.experimental import pallas as pl
from jax.experimental.pallas import tpu as pltpu

_P = 0.0005
_FMIN = float(jnp.finfo(jnp.float32).min)


def _desc_rank(w):
    r, n = w.shape
    wd = w[:, :, None]
    we = w[:, None, :]
    d_idx = jax.lax.broadcasted_iota(jnp.int32, (r, n, n), 1)
    e_idx = jax.lax.broadcasted_iota(jnp.int32, (r, n, n), 2)
    beats = (we > wd) | ((we == wd) & (e_idx < d_idx))
    return jnp.sum(beats.astype(jnp.int32), axis=2)


def _stats_body(feat_ref, x_ref, xf_ref, w_ref, b_ref,
                out_ref, pm_ref, ps_ref, pmf_ref, psf_ref,
                *, n_blocks, blk, c, b_sz):
    j = pl.program_id(0)

    def _stats(x, pm_r, ps_r):
        xr = x.reshape(b_sz, blk // 128, 128)
        bm = jnp.max(xr, axis=1)
        bs = jnp.sum(jnp.exp(xr - bm[:, None, :]), axis=1)
        pm_r[0] = bm
        ps_r[0] = bs

    rem = c - (n_blocks - 1) * blk
    if rem == blk:
        _stats(x_ref[...], pm_ref, ps_ref)
        _stats(xf_ref[...], pmf_ref, psf_ref)
    else:
        @pl.when(j != n_blocks - 1)
        def _full():
            _stats(x_ref[...], pm_ref, ps_ref)
            _stats(xf_ref[...], pmf_ref, psf_ref)

        @pl.when(j == n_blocks - 1)
        def _ragged():
            valid = jax.lax.broadcasted_iota(jnp.int32, (b_sz, blk), 1) < rem
            _stats(jnp.where(valid, x_ref[...], _FMIN), pm_ref, ps_ref)
            _stats(jnp.where(valid, xf_ref[...], _FMIN), pmf_ref, psf_ref)

    y = jax.lax.dot_general(feat_ref[...], w_ref[...],
                            (((1,), (1,)), ((), ())),
                            preferred_element_type=jnp.float32)
    out_ref[...] = y + b_ref[...]


def _fix_body(feat_ref, w16_ref, b_ref, x_ref, xf_ref,
              pm_ref, ps_ref, pmf_ref, psf_ref, prev_ref, out_ref,
              *, k, b_sz):
    def _finish(pm_r, ps_r, x0):
        pm = pm_r[...]
        m2 = jnp.max(pm, axis=0)
        s2 = jnp.sum(ps_r[...] * jnp.exp(pm - m2[None, :, :]), axis=0)
        m_row = jnp.max(m2, axis=1, keepdims=True)
        s_row = jnp.sum(s2 * jnp.exp(m2 - m_row), axis=1, keepdims=True)
        return jnp.exp(x0 - m_row) / s_row

    h = _finish(pm_ref, ps_ref, x_ref[:, 0:1])
    hf = _finish(pmf_ref, psf_ref, xf_ref[:, 0:1])

    eye = (jax.lax.broadcasted_iota(jnp.int32, (b_sz, b_sz), 0)
           == jax.lax.broadcasted_iota(jnp.int32, (b_sz, b_sz), 1))

    def _trow(col):
        return jnp.sum(jnp.where(eye, col, 0), axis=0, keepdims=True)

    def _tcol(row):
        return jnp.sum(jnp.where(eye, row, 0), axis=1, keepdims=True)

    rank_h = _desc_rank(_trow(h))
    rank_hf = _desc_rank(_trow(hf))
    pair = rank_hf == _tcol(rank_h)

    w16 = w16_ref[...]
    rd = _desc_rank(w16)
    w_src = jnp.sum(jnp.where(pair[:, :, None], w16[None, :, :], 0.0), axis=1)
    r_src = jnp.sum(jnp.where(pair[:, :, None], rd[None, :, :], 0), axis=1)
    take = r_src[:, None, :] == rd[:, :, None]
    newval = jnp.sum(jnp.where(take, w_src[:, None, :], 0.0), axis=2)
    w16_mod = jnp.where(rd < k, newval, w16)

    b16 = b_ref[:, 0:b_sz]
    b16_mod = _trow(jnp.sum(jnp.where(pair, b16, 0.0),
                            axis=1, keepdims=True))

    y16 = jax.lax.dot_general(feat_ref[...], w16_mod,
                              (((1,), (1,)), ((), ())),
                              preferred_element_type=jnp.float32)
    out_ref[...] = prev_ref[...]
    out_ref[:, 0:b_sz] = y16 + b16_mod


def kernel(features, features_f, output, output_f, weight_matrix, bias):
    del features_f
    b_sz, d = features.shape
    c = weight_matrix.shape[0]
    k = int(round(c * _P))
    blk = 8192
    n_blocks = pl.cdiv(c, blk)
    bias2 = bias.reshape(1, c)

    stats_body = functools.partial(_stats_body, n_blocks=n_blocks, blk=blk,
                                   c=c, b_sz=b_sz)
    stat_shape = jax.ShapeDtypeStruct((n_blocks, b_sz, 128), jnp.float32)
    stat_spec = pl.BlockSpec((1, b_sz, 128), lambda i: (i, 0, 0))
    out_main, pm, ps, pmf, psf = pl.pallas_call(
        stats_body,
        grid=(n_blocks,),
        in_specs=[
            pl.BlockSpec((b_sz, d), lambda i: (0, 0)),
            pl.BlockSpec((b_sz, blk), lambda i: (0, i)),
            pl.BlockSpec((b_sz, blk), lambda i: (0, i)),
            pl.BlockSpec((blk, d), lambda i: (i, 0)),
            pl.BlockSpec((1, blk), lambda i: (0, i)),
        ],
        out_specs=[
            pl.BlockSpec((b_sz, blk), lambda i: (0, i)),
            stat_spec, stat_spec, stat_spec, stat_spec,
        ],
        out_shape=[jax.ShapeDtypeStruct((b_sz, c), jnp.float32),
                   stat_shape, stat_shape, stat_shape, stat_shape],
        compiler_params=pltpu.CompilerParams(
            dimension_semantics=("parallel",)),
    )(features, output, output_f, weight_matrix, bias2)

    fix_body = functools.partial(_fix_body, k=k, b_sz=b_sz)
    full3 = pl.BlockSpec((n_blocks, b_sz, 128), lambda i: (0, 0, 0))
    head = lambda shape: pl.BlockSpec(shape, lambda i: (0, 0))
    return pl.pallas_call(
        fix_body,
        grid=(1,),
        in_specs=[
            head((b_sz, d)),
            head((b_sz, d)),
            head((1, 128)),
            head((b_sz, 128)),
            head((b_sz, 128)),
            full3, full3, full3, full3,
            head((b_sz, 128)),
        ],
        out_specs=head((b_sz, 128)),
        out_shape=jax.ShapeDtypeStruct((b_sz, c), jnp.float32),
        input_output_aliases={9: 0},
    )(features, weight_matrix, bias2, output, output_f,
      pm, ps, pmf, psf, out_main)
